# Initial kernel scaffold; baseline (speedup 1.0000x reference)
import jax
import jax.numpy as jnp
from jax.experimental import pallas as pl


def kernel(indices, table):
    raise NotImplementedError("write your pallas kernel here")


# speedup vs baseline: 1.7630x; 1.7630x over previous
import functools

import jax
import jax.numpy as jnp
from jax import lax
from jax.experimental import pallas as pl
from jax.experimental.pallas import tpu as pltpu
from jax.experimental.pallas import tpu_sc as plsc

B = 16384
L = 20
D = 32
NC = 2
NS = 16
NW = NC * NS
RPW = B // NW
CH = 64
NCHUNK = RPW // CH
LANES = 16

_mesh = plsc.VectorSubcoreMesh(
    core_axis_name="c", subcore_axis_name="s", num_cores=NC, num_subcores=NS
)


@functools.partial(
    pl.kernel,
    out_type=jax.ShapeDtypeStruct((B, D), jnp.float32),
    mesh=_mesh,
    compiler_params=pltpu.CompilerParams(
        needs_layout_passes=False, use_tc_tiling_on_sc=False),
    scratch_types=[
        pltpu.VMEM((L, RPW), jnp.int32),
        pltpu.VMEM((2, L, CH, D), jnp.float32),
        pltpu.VMEM((CH, D), jnp.float32),
        pltpu.VMEM((CH,), jnp.float32),
        pltpu.VMEM((CH,), jnp.float32),
        pltpu.VMEM((1, D), jnp.float32),
        pltpu.SemaphoreType.DMA,
        pltpu.SemaphoreType.DMA,
    ],
)
def _emb_kernel(table_hbm, idxw_hbm, out_hbm,
                idx_v, g_v, out_v, npad_v, rec_v, t0_v, sem0, sem1):
    wid = lax.axis_index("s") * NC + lax.axis_index("c")
    base = wid * RPW

    pltpu.sync_copy(idxw_hbm.at[wid], idx_v)
    pltpu.sync_copy(table_hbm.at[pl.ds(0, 1)], t0_v)

    sems = (sem0, sem1)

    def fire(c, buf):
        descs = []
        for j in range(L):
            d = pltpu.make_async_copy(
                table_hbm.at[idx_v.at[j, pl.ds(c * CH, CH)]],
                g_v.at[buf, j],
                sems[buf],
            )
            d.start()
            descs.append(d)
        return descs

    pending = fire(0, 0)
    t00 = t0_v[0, 0:16]
    t01 = t0_v[0, 16:32]

    for c in range(NCHUNK):
        buf = c % 2
        nxt = None
        if c + 1 < NCHUNK:
            nxt = fire(c + 1, 1 - buf)

        for rb in range(CH // LANES):
            cnt = jnp.zeros((LANES,), jnp.float32)
            for j in range(L):
                iv = idx_v[j, pl.ds(c * CH + rb * LANES, LANES)]
                cnt = cnt + jnp.where(iv == 0,
                                      jnp.float32(1.0), jnp.float32(0.0))
            npad_v[pl.ds(rb * LANES, LANES)] = cnt
            valid = jnp.maximum(jnp.float32(L) - cnt, jnp.float32(1.0))
            rec_v[pl.ds(rb * LANES, LANES)] = jnp.float32(1.0) / valid

        for d in pending:
            d.wait()

        def row_body(r, carry):
            a0 = g_v[buf, 0, r, 0:16]
            a1 = g_v[buf, 0, r, 16:32]
            for j in range(1, L):
                a0 = a0 + g_v[buf, j, r, 0:16]
                a1 = a1 + g_v[buf, j, r, 16:32]
            r0 = jnp.bitwise_and(r, jnp.int32(-LANES))
            m = lax.iota(jnp.int32, LANES) == (r - r0)
            cv = npad_v[pl.ds(r0, LANES)]
            rv = rec_v[pl.ds(r0, LANES)]
            zero = jnp.zeros((LANES,), jnp.float32)
            np_b = jnp.broadcast_to(jnp.sum(jnp.where(m, cv, zero)), (LANES,))
            rc_b = jnp.broadcast_to(jnp.sum(jnp.where(m, rv, zero)), (LANES,))
            out_v[r, 0:16] = (a0 - np_b * t00) * rc_b
            out_v[r, 16:32] = (a1 - np_b * t01) * rc_b
            return carry

        lax.fori_loop(0, CH, row_body, 0)

        pltpu.sync_copy(out_v, out_hbm.at[pl.ds(base + c * CH, CH)])
        pending = nxt


@jax.jit
def kernel(indices, table):
    idx_w = indices.reshape(NW, RPW, L).transpose(0, 2, 1)
    return _emb_kernel(table, idx_w)
